# speedup vs baseline: 1.1261x; 1.1261x over previous
import functools

import jax
import jax.numpy as jnp
from jax import lax
from jax.experimental import pallas as pl
from jax.experimental.pallas import tpu as pltpu
from jax.experimental.pallas import tpu_sc as plsc

_B, _H, _W = 8, 512, 512
_HW = _H * _W
_N_MATCHED = 4096
_N_UNMATCHED = 1024

_NW = 32
_CHUNK = 128
_LANES = 16

_N_M = 2 * _B * _N_MATCHED
_N_U = _B * _N_UNMATCHED
_M_PER_W = _N_M // _NW
_U_PER_W = _N_U // _NW
_M_CHUNKS = _M_PER_W // _CHUNK
_U_CHUNKS = _U_PER_W // _CHUNK


def _sigmoid16(x):
    return 1.0 / (1.0 + jnp.exp(-x))


@functools.partial(
    pl.kernel,
    out_type=jax.ShapeDtypeStruct((_NW, _LANES), jnp.float32),
    mesh=plsc.VectorSubcoreMesh(core_axis_name="c", subcore_axis_name="s"),
    scratch_types=[
        pltpu.VMEM((_CHUNK,), jnp.int32),
        pltpu.VMEM((_CHUNK,), jnp.int32),
        pltpu.VMEM((_CHUNK,), jnp.float32),
        pltpu.VMEM((_CHUNK,), jnp.float32),
        pltpu.VMEM((_LANES,), jnp.float32),
        pltpu.SemaphoreType.DMA,
    ],
)
def _bm_loss_sc(inp_hbm, tgt_hbm, am_hbm, bm_hbm, au_hbm, bu_hbm, out_hbm,
                ia_v, ib_v, va_v, vb_v, acc_v, sem):
    wid = lax.axis_index("s") * 2 + lax.axis_index("c")
    acc_v[...] = jnp.zeros((_LANES,), jnp.float32)

    def matched_step(i, carry):
        base = wid * _M_PER_W + i * _CHUNK
        pltpu.sync_copy(am_hbm.at[pl.ds(base, _CHUNK)], ia_v)
        pltpu.sync_copy(bm_hbm.at[pl.ds(base, _CHUNK)], ib_v)
        cpa = pltpu.async_copy(inp_hbm.at[ia_v], va_v, sem)
        cpb = pltpu.async_copy(tgt_hbm.at[ib_v], vb_v, sem)
        cpa.wait()
        cpb.wait()
        for j in range(_CHUNK // _LANES):
            sl = pl.ds(j * _LANES, _LANES)
            d = _sigmoid16(va_v[sl]) - vb_v[sl]
            acc_v[...] += 2.0 * (d * d)
        return carry

    lax.fori_loop(0, _M_CHUNKS, matched_step, 0)

    def unmatched_step(i, carry):
        base = wid * _U_PER_W + i * _CHUNK
        pltpu.sync_copy(au_hbm.at[pl.ds(base, _CHUNK)], ia_v)
        pltpu.sync_copy(bu_hbm.at[pl.ds(base, _CHUNK)], ib_v)
        cpa = pltpu.async_copy(inp_hbm.at[ia_v], va_v, sem)
        cpb = pltpu.async_copy(inp_hbm.at[ib_v], vb_v, sem)
        cpa.wait()
        cpb.wait()
        for j in range(_CHUNK // _LANES):
            sl = pl.ds(j * _LANES, _LANES)
            d = _sigmoid16(va_v[sl]) - _sigmoid16(vb_v[sl])
            acc_v[...] += d * d
        return carry

    lax.fori_loop(0, _U_CHUNKS, unmatched_step, 0)

    pltpu.sync_copy(acc_v, out_hbm.at[wid])


def kernel(input, target, pred_birth_idx, pred_death_idx, tgt_birth_idx,
           tgt_death_idx, unm_birth_idx, unm_death_idx):
    inp_flat = input.reshape(_B * _HW)
    tgt_flat = target.reshape(_B * _HW)
    offs = (jnp.arange(_B, dtype=jnp.int32) * _HW)[:, None]

    def flat(idx):
        return (idx.astype(jnp.int32) + offs).reshape(-1)

    am = jnp.concatenate([flat(pred_birth_idx), flat(pred_death_idx)])
    bm = jnp.concatenate([flat(tgt_birth_idx), flat(tgt_death_idx)])
    au = flat(unm_birth_idx)
    bu = flat(unm_death_idx)

    partials = _bm_loss_sc(inp_flat, tgt_flat, am, bm, au, bu)
    return jnp.sum(partials).reshape(1)


# speedup vs baseline: 1.7397x; 1.5449x over previous
import functools

import jax
import jax.numpy as jnp
from jax import lax
from jax.experimental import pallas as pl
from jax.experimental.pallas import tpu as pltpu
from jax.experimental.pallas import tpu_sc as plsc

_B, _H, _W = 8, 512, 512
_HW = _H * _W
_N_MATCHED = 4096
_N_UNMATCHED = 1024

_NW = 32
_CHUNK = 128
_LANES = 16

_N_M = 2 * _B * _N_MATCHED
_N_U = _B * _N_UNMATCHED
_M_PER_W = _N_M // _NW
_U_PER_W = _N_U // _NW
_PER_W = _M_PER_W + _U_PER_W
_M_CHUNKS = _M_PER_W // _CHUNK
_U_CHUNKS = _U_PER_W // _CHUNK


def _sigmoid16(x):
    return 1.0 / (1.0 + jnp.exp(-x))


@functools.partial(
    pl.kernel,
    out_type=jax.ShapeDtypeStruct((_NW, _LANES), jnp.float32),
    mesh=plsc.VectorSubcoreMesh(core_axis_name="c", subcore_axis_name="s"),
    scratch_types=[
        pltpu.VMEM((_PER_W,), jnp.int32),
        pltpu.VMEM((_PER_W,), jnp.int32),
        pltpu.VMEM((2 * _PER_W,), jnp.float32),
        pltpu.SemaphoreType.DMA,
        pltpu.SemaphoreType.DMA,
    ],
)
def _bm_loss_sc(inp_hbm, tgt_hbm, am_hbm, bm_hbm, au_hbm, bu_hbm, out_hbm,
                ia_v, ib_v, vab_v, isem, gsem):
    wid = lax.axis_index("s") * 2 + lax.axis_index("c")

    c1 = pltpu.async_copy(am_hbm.at[pl.ds(wid * _M_PER_W, _M_PER_W)],
                          ia_v.at[pl.ds(0, _M_PER_W)], isem)
    c2 = pltpu.async_copy(bm_hbm.at[pl.ds(wid * _M_PER_W, _M_PER_W)],
                          ib_v.at[pl.ds(0, _M_PER_W)], isem)
    c3 = pltpu.async_copy(au_hbm.at[pl.ds(wid * _U_PER_W, _U_PER_W)],
                          ia_v.at[pl.ds(_M_PER_W, _U_PER_W)], isem)
    c4 = pltpu.async_copy(bu_hbm.at[pl.ds(wid * _U_PER_W, _U_PER_W)],
                          ib_v.at[pl.ds(_M_PER_W, _U_PER_W)], isem)
    c1.wait()
    c2.wait()
    c3.wait()
    c4.wait()

    def fire_matched(i, carry):
        sl = pl.ds(i * _CHUNK, _CHUNK)
        pltpu.async_copy(inp_hbm.at[ia_v.at[sl]], vab_v.at[sl], gsem)
        pltpu.async_copy(tgt_hbm.at[ib_v.at[sl]],
                         vab_v.at[pl.ds(_PER_W + i * _CHUNK, _CHUNK)], gsem)
        return carry

    lax.fori_loop(0, _M_CHUNKS, fire_matched, 0)
    for j in range(_M_CHUNKS, _M_CHUNKS + _U_CHUNKS):
        sl = pl.ds(j * _CHUNK, _CHUNK)
        pltpu.async_copy(inp_hbm.at[ia_v.at[sl]], vab_v.at[sl], gsem)
        pltpu.async_copy(inp_hbm.at[ib_v.at[sl]],
                         vab_v.at[pl.ds(_PER_W + j * _CHUNK, _CHUNK)], gsem)

    pltpu.make_async_copy(inp_hbm.at[pl.ds(0, 2 * _PER_W)], vab_v, gsem).wait()

    def matched_body(k, acc):
        a = vab_v[pl.ds(k * _LANES, _LANES)]
        b = vab_v[pl.ds(_PER_W + k * _LANES, _LANES)]
        d = _sigmoid16(a) - b
        return acc + 2.0 * (d * d)

    acc = lax.fori_loop(0, _M_PER_W // _LANES, matched_body,
                        jnp.zeros((_LANES,), jnp.float32))

    def unm_body(k, acc):
        a = vab_v[pl.ds(k * _LANES, _LANES)]
        b = vab_v[pl.ds(_PER_W + k * _LANES, _LANES)]
        d = _sigmoid16(a) - _sigmoid16(b)
        return acc + d * d

    acc = lax.fori_loop(_M_PER_W // _LANES, _PER_W // _LANES, unm_body, acc)

    pl.run_scoped(
        lambda acc_ref: (acc_ref.__setitem__((...,), acc),
                         pltpu.sync_copy(acc_ref, out_hbm.at[wid])),
        pltpu.VMEM((_LANES,), jnp.float32),
    )


def kernel(input, target, pred_birth_idx, pred_death_idx, tgt_birth_idx,
           tgt_death_idx, unm_birth_idx, unm_death_idx):
    inp_flat = input.reshape(_B * _HW)
    tgt_flat = target.reshape(_B * _HW)
    offs = (jnp.arange(_B, dtype=jnp.int32) * _HW)[:, None]

    def flat(idx):
        return (idx.astype(jnp.int32) + offs).reshape(-1)

    am = jnp.concatenate([flat(pred_birth_idx), flat(pred_death_idx)])
    bm = jnp.concatenate([flat(tgt_birth_idx), flat(tgt_death_idx)])
    au = flat(unm_birth_idx)
    bu = flat(unm_death_idx)

    partials = _bm_loss_sc(inp_flat, tgt_flat, am, bm, au, bu)
    return jnp.sum(partials).reshape(1)
